# Initial kernel scaffold; baseline (speedup 1.0000x reference)
import jax
import jax.numpy as jnp
from jax.experimental import pallas as pl


def kernel(text, offsets, table, fc_w, fc_b):
    raise NotImplementedError("write your pallas kernel here")


# speedup vs baseline: 29.1593x; 29.1593x over previous
import functools

import jax
import jax.numpy as jnp
from jax import lax
from jax.experimental import pallas as pl
from jax.experimental.pallas import tpu as pltpu
from jax.experimental.pallas import tpu_sc as plsc

EMBED = 64
NC, NS, L = 2, 16, 16
NW = NC * NS
CHUNK = 128


def _bag_sums_sc(text_pad, offs_ext, table, B):
    b_per_w = B // NW
    acc_rows = b_per_w + 8
    bs_steps = max(1, (b_per_w - 1).bit_length())
    mesh = plsc.VectorSubcoreMesh(core_axis_name="c", subcore_axis_name="s",
                                  num_cores=NC, num_subcores=NS)

    @functools.partial(
        pl.kernel,
        out_type=jax.ShapeDtypeStruct((B, EMBED), jnp.float32),
        mesh=mesh,
        scratch_types=[
            pltpu.VMEM((acc_rows,), jnp.int32),
            pltpu.VMEM((CHUNK,), jnp.int32),
            pltpu.VMEM((CHUNK,), jnp.int32),
            pltpu.VMEM((CHUNK, EMBED), jnp.float32),
            pltpu.VMEM_SHARED((NS * acc_rows, EMBED), jnp.float32),
            pltpu.SemaphoreType.DMA,
        ],
        compiler_params=pltpu.CompilerParams(
            needs_layout_passes=False, use_tc_tiling_on_sc=False),
    )
    def k(text_hbm, offs_hbm, table_hbm, out_hbm,
          offs_v, idx_v, seg_v, rows_v, acc_sp, sem):
        cid = lax.axis_index("c")
        sid = lax.axis_index("s")
        wid = sid * NC + cid
        bag0 = pl.multiple_of(wid * b_per_w, 8)
        slot0 = sid * acc_rows

        pltpu.sync_copy(offs_hbm.at[pl.ds(bag0, acc_rows)], offs_v)

        zero = jnp.zeros((L,), jnp.float32)
        zrows = min(CHUNK, acc_rows)
        def zbody(r, c):
            for kk in range(EMBED // L):
                rows_v[r, pl.ds(kk * L, L)] = zero
            return c
        lax.fori_loop(0, zrows, zbody, 0)
        off = 0
        while off < acc_rows:
            w = min(zrows, acc_rows - off)
            pltpu.sync_copy(rows_v.at[pl.ds(0, w)],
                            acc_sp.at[pl.ds(slot0 + off, w)])
            off += w

        v_head = offs_v[pl.ds(0, L)]
        t_start = v_head[0]
        v_tail = offs_v[pl.ds(b_per_w - 8, L)]
        t_end = v_tail[8]
        s0 = t_start - lax.rem(t_start, 8)
        n_chunks = lax.div(t_end - s0 + (CHUNK - 1), CHUNK)

        lanes = lax.iota(jnp.int32, L)

        def chunk_body(j, c):
            s = pl.multiple_of(s0 + j * CHUNK, 8)
            pltpu.sync_copy(text_hbm.at[pl.ds(s, CHUNK)], idx_v)
            cp = pltpu.async_copy(table_hbm.at[idx_v], rows_v, sem)

            def seg_body(g, cc):
                t = s + g * L + lanes
                valid = (t >= t_start) & (t < t_end)
                lo = jnp.zeros((L,), jnp.int32)
                hi = jnp.full((L,), b_per_w, jnp.int32)
                for _ in range(bs_steps):
                    mid = (lo + hi) >> 1
                    v = plsc.load_gather(offs_v, [mid])
                    pred = v <= t
                    lo = jnp.where(pred, mid, lo)
                    hi = jnp.where(pred, hi, mid)
                seg = slot0 + jnp.where(valid, lo, jnp.int32(b_per_w))
                seg_v[pl.ds(g * L, L)] = seg
                return cc

            lax.fori_loop(0, CHUNK // L, seg_body, 0)
            cp.wait()
            pltpu.sync_copy(rows_v, acc_sp.at[seg_v], add=True)
            return c

        lax.fori_loop(0, n_chunks, chunk_body, 0)

        pltpu.sync_copy(acc_sp.at[pl.ds(slot0, b_per_w)], rows_v)
        pltpu.sync_copy(rows_v, out_hbm.at[pl.ds(bag0, b_per_w)])

    return k(text_pad, offs_ext, table)


def _linear_tc(sums, offs, offs_next, fc_wt, fc_b_row):
    def body(sums_ref, offs_ref, offsn_ref, w_ref, b_ref, out_ref):
        counts = (offsn_ref[...] - offs_ref[...]).astype(jnp.float32)
        inv = 1.0 / jnp.maximum(counts, 1.0)
        pooled = sums_ref[...] * inv
        out_ref[...] = (
            jnp.dot(pooled, w_ref[...], preferred_element_type=jnp.float32)
            + b_ref[...]
        )

    B = sums.shape[0]
    n = fc_wt.shape[1]
    return pl.pallas_call(
        body,
        out_shape=jax.ShapeDtypeStruct((B, n), jnp.float32),
    )(sums, offs, offs_next, fc_wt, fc_b_row)


def kernel(text, offsets, table, fc_w, fc_b):
    T = text.shape[0]
    B = offsets.shape[0]
    text_pad = jnp.concatenate(
        [text.astype(jnp.int32), jnp.zeros((CHUNK + 8,), jnp.int32)])
    offs32 = offsets.astype(jnp.int32)
    offs_ext = jnp.concatenate([offs32, jnp.full((8,), T, jnp.int32)])
    sums = _bag_sums_sc(text_pad, offs_ext, table, B)
    offs_next = jnp.concatenate([offs32[1:], jnp.full((1,), T, jnp.int32)])
    out = _linear_tc(sums, offs32[:, None], offs_next[:, None],
                     fc_w.T, fc_b[None, :])
    return out
